# Initial kernel scaffold; baseline (speedup 1.0000x reference)
import functools

import jax
import jax.numpy as jnp
from jax import lax
from jax.experimental import pallas as pl
from jax.experimental.pallas import tpu as pltpu
from jax.experimental.pallas import tpu_sc as plsc

VOCAB = 1000000
EMBED = 32
B = 16384
L = 50
N = B * L

NC = 2
NS = 16
NW = NC * NS
N_PER_W = N // NW
CHUNK = 1024
NSTEP = N_PER_W // CHUNK


@functools.partial(
    pl.kernel,
    out_type=jax.ShapeDtypeStruct((N, EMBED), jnp.float32),
    mesh=plsc.VectorSubcoreMesh(core_axis_name="c", subcore_axis_name="s"),
    scratch_types=[
        pltpu.VMEM((CHUNK,), jnp.int32),
        pltpu.VMEM((CHUNK, EMBED), jnp.float32),
        pltpu.SemaphoreType.DMA,
    ],
)
def _gather_sc(idx_hbm, table_hbm, out_hbm, idx_v, rows_v, sem):
    wid = lax.axis_index("s") * NC + lax.axis_index("c")
    base = wid * N_PER_W

    def step(i, carry):
        off = base + i * CHUNK
        pltpu.sync_copy(idx_hbm.at[pl.ds(off, CHUNK)], idx_v)
        pltpu.async_copy(table_hbm.at[idx_v], rows_v, sem).wait()
        pltpu.sync_copy(rows_v, out_hbm.at[pl.ds(off, CHUNK)])
        return carry

    lax.fori_loop(0, NSTEP, step, 0)


def kernel(x, embeddings):
    idx = x.reshape(N)
    out = _gather_sc(idx, embeddings)
    return out.reshape(B, L, EMBED)


# speedup vs baseline: 1.0943x; 1.0943x over previous
import functools

import jax
import jax.numpy as jnp
from jax import lax
from jax.experimental import pallas as pl
from jax.experimental.pallas import tpu as pltpu
from jax.experimental.pallas import tpu_sc as plsc

VOCAB = 1000000
EMBED = 32
B = 16384
L = 50
N = B * L

NC = 2
NS = 16
NW = NC * NS
N_PER_W = N // NW
CHUNK = 1024
NSTEP = N_PER_W // CHUNK


@functools.partial(
    pl.kernel,
    out_type=jax.ShapeDtypeStruct((N, EMBED), jnp.float32),
    mesh=plsc.VectorSubcoreMesh(core_axis_name="c", subcore_axis_name="s"),
    scratch_types=[
        pltpu.VMEM((CHUNK,), jnp.int32),
        pltpu.VMEM((CHUNK, EMBED), jnp.float32),
        pltpu.SemaphoreType.DMA,
    ],
    compiler_params=pltpu.CompilerParams(use_tc_tiling_on_sc=False),
)
def _gather_sc(idx_hbm, table_hbm, out_hbm, idx_v, rows_v, sem):
    wid = lax.axis_index("s") * NC + lax.axis_index("c")
    base = wid * N_PER_W

    def step(i, carry):
        off = base + i * CHUNK
        pltpu.sync_copy(idx_hbm.at[pl.ds(off, CHUNK)], idx_v)
        pltpu.async_copy(table_hbm.at[idx_v], rows_v, sem).wait()
        pltpu.sync_copy(rows_v, out_hbm.at[pl.ds(off, CHUNK)])
        return carry

    lax.fori_loop(0, NSTEP, step, 0)


def kernel(x, embeddings):
    idx = x.reshape(N)
    out = _gather_sc(idx, embeddings)
    return out.reshape(B, L, EMBED)


# speedup vs baseline: 1.1114x; 1.0156x over previous
import functools

import jax
import jax.numpy as jnp
from jax import lax
from jax.experimental import pallas as pl
from jax.experimental.pallas import tpu as pltpu
from jax.experimental.pallas import tpu_sc as plsc

VOCAB = 1000000
EMBED = 32
B = 16384
L = 50
N = B * L

NC = 2
NS = 16
NW = NC * NS
N_PER_W = N // NW
CHUNK = 1024
NSTEP = N_PER_W // CHUNK
NBUF = 3


@functools.partial(
    pl.kernel,
    out_type=jax.ShapeDtypeStruct((N, EMBED), jnp.float32),
    mesh=plsc.VectorSubcoreMesh(core_axis_name="c", subcore_axis_name="s"),
    scratch_types=[
        pltpu.VMEM((NSTEP, CHUNK), jnp.int32),
        [pltpu.VMEM((CHUNK, EMBED), jnp.float32) for _ in range(NBUF)],
        [pltpu.SemaphoreType.DMA for _ in range(NBUF)],
        [pltpu.SemaphoreType.DMA for _ in range(NBUF)],
    ],
    compiler_params=pltpu.CompilerParams(use_tc_tiling_on_sc=False),
)
def _gather_sc(idx_hbm, table_hbm, out_hbm, idx_v, rows_v, g_sems, o_sems):
    wid = lax.axis_index("s") * NC + lax.axis_index("c")
    base = wid * N_PER_W

    pltpu.sync_copy(idx_hbm.at[wid], idx_v)

    gather_d = [None] * NSTEP
    out_d = [None] * NSTEP
    for i in range(NSTEP):
        b = i % NBUF
        if i >= NBUF:
            out_d[i - NBUF].wait()
        gather_d[i] = pltpu.async_copy(
            table_hbm.at[idx_v.at[i]], rows_v[b], g_sems[b])
        if i >= 1:
            j = i - 1
            gather_d[j].wait()
            out_d[j] = pltpu.async_copy(
                rows_v[j % NBUF],
                out_hbm.at[pl.ds(base + j * CHUNK, CHUNK)],
                o_sems[j % NBUF])
    j = NSTEP - 1
    gather_d[j].wait()
    out_d[j] = pltpu.async_copy(
        rows_v[j % NBUF],
        out_hbm.at[pl.ds(base + j * CHUNK, CHUNK)],
        o_sems[j % NBUF])
    for j in range(max(0, NSTEP - NBUF), NSTEP):
        out_d[j].wait()


def kernel(x, embeddings):
    idx = x.reshape(NW, NSTEP, CHUNK)
    out = _gather_sc(idx, embeddings)
    return out.reshape(B, L, EMBED)


# speedup vs baseline: 1.1125x; 1.0010x over previous
import functools

import jax
import jax.numpy as jnp
from jax import lax
from jax.experimental import pallas as pl
from jax.experimental.pallas import tpu as pltpu
from jax.experimental.pallas import tpu_sc as plsc

VOCAB = 1000000
EMBED = 32
B = 16384
L = 50
N = B * L

NC = 2
NS = 16
NW = NC * NS
N_PER_W = N // NW
CHUNK = 512
NSTEP = N_PER_W // CHUNK
NBUF = 6
SKEW = 3


@functools.partial(
    pl.kernel,
    out_type=jax.ShapeDtypeStruct((N, EMBED), jnp.float32),
    mesh=plsc.VectorSubcoreMesh(core_axis_name="c", subcore_axis_name="s"),
    scratch_types=[
        pltpu.VMEM((NSTEP, CHUNK), jnp.int32),
        [pltpu.VMEM((CHUNK, EMBED), jnp.float32) for _ in range(NBUF)],
        [pltpu.SemaphoreType.DMA for _ in range(NBUF)],
        [pltpu.SemaphoreType.DMA for _ in range(NBUF)],
    ],
    compiler_params=pltpu.CompilerParams(use_tc_tiling_on_sc=False),
)
def _gather_sc(idx_hbm, table_hbm, out_hbm, idx_v, rows_v, g_sems, o_sems):
    wid = lax.axis_index("s") * NC + lax.axis_index("c")
    base = wid * N_PER_W

    pltpu.sync_copy(idx_hbm.at[wid], idx_v)

    gather_d = [None] * NSTEP
    out_d = [None] * NSTEP

    def drain(j):
        gather_d[j].wait()
        out_d[j] = pltpu.async_copy(
            rows_v[j % NBUF],
            out_hbm.at[pl.ds(base + j * CHUNK, CHUNK)],
            o_sems[j % NBUF])

    for i in range(NSTEP):
        b = i % NBUF
        if i >= NBUF:
            out_d[i - NBUF].wait()
        gather_d[i] = pltpu.async_copy(
            table_hbm.at[idx_v.at[i]], rows_v[b], g_sems[b])
        if i >= SKEW:
            drain(i - SKEW)
    for j in range(max(0, NSTEP - SKEW), NSTEP):
        drain(j)
    for j in range(max(0, NSTEP - NBUF), NSTEP):
        out_d[j].wait()


def kernel(x, embeddings):
    idx = x.reshape(NW, NSTEP, CHUNK)
    out = _gather_sc(idx, embeddings)
    return out.reshape(B, L, EMBED)


# speedup vs baseline: 1.4299x; 1.2853x over previous
import functools

import jax
import jax.numpy as jnp
from jax import lax
from jax.experimental import pallas as pl
from jax.experimental.pallas import tpu as pltpu
from jax.experimental.pallas import tpu_sc as plsc

VOCAB = 1000000
EMBED = 32
B = 16384
L = 50

NC = 2
NS = 16
NW = NC * NS
B_PER_W = B // NW
BB = 16
NSTEP = B_PER_W // BB
CHUNK = BB * L


def _transpose_block(rows, outst):
    iota = lax.iota(jnp.int32, 16)
    iota_l = iota * L

    def tl(l, c):
        rowv = iota_l + l
        for j in range(EMBED):
            v = plsc.load_gather(rows, [rowv, jnp.full((16,), j, jnp.int32)])
            outst[l, j, :] = v
        return c

    lax.fori_loop(0, L, tl, 0)


@functools.partial(
    pl.kernel,
    out_type=jax.ShapeDtypeStruct((L, EMBED, B), jnp.float32),
    mesh=plsc.VectorSubcoreMesh(core_axis_name="c", subcore_axis_name="s"),
    scratch_types=[
        pltpu.VMEM((NSTEP, CHUNK), jnp.int32),
        pltpu.VMEM((CHUNK, EMBED), jnp.float32),
        pltpu.VMEM((CHUNK, EMBED), jnp.float32),
        pltpu.VMEM((L, EMBED, BB), jnp.float32),
        pltpu.SemaphoreType.DMA,
        pltpu.SemaphoreType.DMA,
        pltpu.SemaphoreType.DMA,
    ],
    compiler_params=pltpu.CompilerParams(
        use_tc_tiling_on_sc=False, needs_layout_passes=False),
)
def _gather_sc(idx_hbm, table_hbm, out_hbm, idx_v, rows0, rows1, outst,
               gsem0, gsem1, osem):
    wid = lax.axis_index("s") * NC + lax.axis_index("c")
    base_b = wid * B_PER_W

    pltpu.sync_copy(idx_hbm.at[wid], idx_v)

    rows = [rows0, rows1]
    gsems = [gsem0, gsem1]

    def out_dst(k):
        return out_hbm.at[:, :, pl.ds(base_b + k * BB, BB)]

    def step(k, rbuf, gsem, first):
        pltpu.make_async_copy(table_hbm.at[idx_v.at[k]], rbuf, gsem).wait()
        if not first:
            pltpu.make_async_copy(outst, out_dst(k - 1), osem).wait()
        _transpose_block(rbuf, outst)
        pltpu.async_copy(outst, out_dst(k), osem)
        @pl.when(k + 2 < NSTEP)
        def _():
            pltpu.async_copy(table_hbm.at[idx_v.at[k + 2]], rbuf, gsem)

    pltpu.async_copy(table_hbm.at[idx_v.at[0]], rows[0], gsems[0])
    pltpu.async_copy(table_hbm.at[idx_v.at[1]], rows[1], gsems[1])

    step(0, rows[0], gsems[0], True)
    step(1, rows[1], gsems[1], False)

    def pair(t, c):
        k0 = t * 2
        step(k0, rows[0], gsems[0], False)
        step(k0 + 1, rows[1], gsems[1], False)
        return c

    lax.fori_loop(1, NSTEP // 2, pair, 0)

    pltpu.make_async_copy(outst, out_dst(NSTEP - 1), osem).wait()


def kernel(x, embeddings):
    idx = x.reshape(NW, NSTEP, CHUNK)
    out_phys = _gather_sc(idx, embeddings)
    return out_phys.transpose(2, 0, 1)


# speedup vs baseline: 1.9881x; 1.3904x over previous
import functools

import jax
import jax.numpy as jnp
from jax import lax
from jax.experimental import pallas as pl
from jax.experimental.pallas import tpu as pltpu
from jax.experimental.pallas import tpu_sc as plsc

VOCAB = 1000000
EMBED = 32
B = 16384
L = 50

NC = 2
NS = 16
NW = NC * NS
B_PER_W = B // NW
BB = 16
NSTEP = B_PER_W // BB
CHUNK = BB * L


def _transpose_block(rows, outst):
    iota = lax.iota(jnp.int32, 16)
    jv0 = iota
    jv1 = iota + 16

    def tl(l, c):
        lv = jnp.full((16,), l, jnp.int32)
        for b_loc in range(BB):
            k = b_loc * L + l
            bv = jnp.full((16,), b_loc, jnp.int32)
            v0 = rows[k, pl.ds(0, 16)]
            v1 = rows[k, pl.ds(16, 16)]
            plsc.store_scatter(outst, [lv, jv0, bv], v0)
            plsc.store_scatter(outst, [lv, jv1, bv], v1)
        return c

    lax.fori_loop(0, L, tl, 0)


@functools.partial(
    pl.kernel,
    out_type=jax.ShapeDtypeStruct((L, EMBED, B), jnp.float32),
    mesh=plsc.VectorSubcoreMesh(core_axis_name="c", subcore_axis_name="s"),
    scratch_types=[
        pltpu.VMEM((NSTEP, CHUNK), jnp.int32),
        pltpu.VMEM((CHUNK, EMBED), jnp.float32),
        pltpu.VMEM((CHUNK, EMBED), jnp.float32),
        pltpu.VMEM((L, EMBED, BB), jnp.float32),
        pltpu.SemaphoreType.DMA,
        pltpu.SemaphoreType.DMA,
        pltpu.SemaphoreType.DMA,
    ],
    compiler_params=pltpu.CompilerParams(
        use_tc_tiling_on_sc=False, needs_layout_passes=False),
)
def _gather_sc(idx_hbm, table_hbm, out_hbm, idx_v, rows0, rows1, outst,
               gsem0, gsem1, osem):
    wid = lax.axis_index("s") * NC + lax.axis_index("c")
    base_b = wid * B_PER_W

    pltpu.sync_copy(idx_hbm.at[wid], idx_v)

    rows = [rows0, rows1]
    gsems = [gsem0, gsem1]

    def out_dst(k):
        return out_hbm.at[:, :, pl.ds(base_b + k * BB, BB)]

    def step(k, rbuf, gsem, first):
        pltpu.make_async_copy(table_hbm.at[idx_v.at[k]], rbuf, gsem).wait()
        if not first:
            pltpu.make_async_copy(outst, out_dst(k - 1), osem).wait()
        _transpose_block(rbuf, outst)
        pltpu.async_copy(outst, out_dst(k), osem)
        @pl.when(k + 2 < NSTEP)
        def _():
            pltpu.async_copy(table_hbm.at[idx_v.at[k + 2]], rbuf, gsem)

    pltpu.async_copy(table_hbm.at[idx_v.at[0]], rows[0], gsems[0])
    pltpu.async_copy(table_hbm.at[idx_v.at[1]], rows[1], gsems[1])

    step(0, rows[0], gsems[0], True)
    step(1, rows[1], gsems[1], False)

    def pair(t, c):
        k0 = t * 2
        step(k0, rows[0], gsems[0], False)
        step(k0 + 1, rows[1], gsems[1], False)
        return c

    lax.fori_loop(1, NSTEP // 2, pair, 0)

    pltpu.make_async_copy(outst, out_dst(NSTEP - 1), osem).wait()


def kernel(x, embeddings):
    idx = x.reshape(NW, NSTEP, CHUNK)
    out_phys = _gather_sc(idx, embeddings)
    return out_phys.transpose(2, 0, 1)


# speedup vs baseline: 2.1712x; 1.0921x over previous
import functools

import jax
import jax.numpy as jnp
from jax import lax
from jax.experimental import pallas as pl
from jax.experimental.pallas import tpu as pltpu
from jax.experimental.pallas import tpu_sc as plsc

VOCAB = 1000000
EMBED = 32
B = 16384
L = 50

NC = 2
NS = 16
NW = NC * NS
B_PER_W = B // NW
BB = 16
NSTEP = B_PER_W // BB
CHUNK = BB * L


def _transpose_block(rows, outst):
    iota = lax.iota(jnp.int32, 16)
    jv0 = iota
    jv1 = iota + 16

    @plsc.parallel_loop(0, L, unroll=2)
    def tl(l):
        lv = jnp.full((16,), l, jnp.int32)
        for b_loc in range(BB):
            k = b_loc * L + l
            bv = jnp.full((16,), b_loc, jnp.int32)
            v0 = rows[k, pl.ds(0, 16)]
            v1 = rows[k, pl.ds(16, 16)]
            plsc.store_scatter(outst, [lv, jv0, bv], v0)
            plsc.store_scatter(outst, [lv, jv1, bv], v1)


@functools.partial(
    pl.kernel,
    out_type=jax.ShapeDtypeStruct((L, EMBED, B), jnp.float32),
    mesh=plsc.VectorSubcoreMesh(core_axis_name="c", subcore_axis_name="s"),
    scratch_types=[
        pltpu.VMEM((NSTEP, CHUNK), jnp.int32),
        pltpu.VMEM((CHUNK, EMBED), jnp.float32),
        pltpu.VMEM((CHUNK, EMBED), jnp.float32),
        pltpu.VMEM((L, EMBED, BB), jnp.float32),
        pltpu.SemaphoreType.DMA,
        pltpu.SemaphoreType.DMA,
        pltpu.SemaphoreType.DMA,
    ],
    compiler_params=pltpu.CompilerParams(
        use_tc_tiling_on_sc=False, needs_layout_passes=False),
)
def _gather_sc(idx_hbm, table_hbm, out_hbm, idx_v, rows0, rows1, outst,
               gsem0, gsem1, osem):
    wid = lax.axis_index("s") * NC + lax.axis_index("c")
    base_b = wid * B_PER_W

    pltpu.sync_copy(idx_hbm.at[wid], idx_v)

    rows = [rows0, rows1]
    gsems = [gsem0, gsem1]

    def out_dst(k):
        return out_hbm.at[:, :, pl.ds(base_b + k * BB, BB)]

    def step(k, rbuf, gsem, first):
        pltpu.make_async_copy(table_hbm.at[idx_v.at[k]], rbuf, gsem).wait()
        if not first:
            pltpu.make_async_copy(outst, out_dst(k - 1), osem).wait()
        _transpose_block(rbuf, outst)
        pltpu.async_copy(outst, out_dst(k), osem)
        @pl.when(k + 2 < NSTEP)
        def _():
            pltpu.async_copy(table_hbm.at[idx_v.at[k + 2]], rbuf, gsem)

    pltpu.async_copy(table_hbm.at[idx_v.at[0]], rows[0], gsems[0])
    pltpu.async_copy(table_hbm.at[idx_v.at[1]], rows[1], gsems[1])

    step(0, rows[0], gsems[0], True)
    step(1, rows[1], gsems[1], False)

    def pair(t, c):
        k0 = t * 2
        step(k0, rows[0], gsems[0], False)
        step(k0 + 1, rows[1], gsems[1], False)
        return c

    lax.fori_loop(1, NSTEP // 2, pair, 0)

    pltpu.make_async_copy(outst, out_dst(NSTEP - 1), osem).wait()


def kernel(x, embeddings):
    idx = x.reshape(NW, NSTEP, CHUNK)
    out_phys = _gather_sc(idx, embeddings)
    return out_phys.transpose(2, 0, 1)
